# baseline (speedup 1.0000x reference)
import functools

import jax
import jax.numpy as jnp
import numpy as np
from jax import lax
from jax.experimental import pallas as pl
from jax.experimental.pallas import tpu as pltpu
from jax.experimental.pallas import tpu_sc as plsc

_N = 2048 * 2048
_LANES = 128
_ROWS = _N // _LANES
_BLK_ROWS = 512
_GRID = _ROWS // _BLK_ROWS
_W_LO = float(np.float32(1e-5))
_W_HI = float(np.float32(1.0 - 1e-5))

_NC, _NS, _L = 2, 16, 16
_NW = _NC * _NS
_CROWS = 128
_WROWS = _ROWS // _NW
_NCHUNKS = _WROWS // _CROWS

_ROW0 = 110
_R1 = 32
_R2 = 512
_TOP0 = _ROW0 * _LANES


def _keys_body(pred_ref, gt_ref, keys_ref):
    p0 = pred_ref[0]
    p1 = pred_ref[1]
    m = gt_ref[...]
    psel = jnp.where(m, p1, p0)
    pt = _W_LO * (p0 + p1) + (_W_HI - _W_LO) * psel + _W_LO
    key = lax.bitcast_convert_type(pt, jnp.int32)
    keys_ref[...] = jnp.where(m, key, 0)


def _keys_call(pred3, gt2):
    blk = (_BLK_ROWS, _LANES)
    return pl.pallas_call(
        _keys_body,
        grid=(_GRID,),
        in_specs=[
            pl.BlockSpec((2, _BLK_ROWS, _LANES), lambda i: (0, i, 0)),
            pl.BlockSpec(blk, lambda i: (i, 0)),
        ],
        out_specs=pl.BlockSpec(blk, lambda i: (i, 0)),
        out_shape=jax.ShapeDtypeStruct((_ROWS, _LANES), jnp.int32),
    )(pred3, gt2)


@functools.lru_cache(maxsize=None)
def _make_hist_kernel(level):
    mesh = plsc.VectorSubcoreMesh(
        core_axis_name="c", subcore_axis_name="s",
        num_cores=_NC, num_subcores=_NS,
    )
    rows = _R1 if level == 1 else _R2
    crows = 256 if level == 1 else _CROWS
    nchunks = _WROWS // crows

    @functools.partial(
        pl.kernel,
        out_type=jax.ShapeDtypeStruct((_NW, rows, _LANES), jnp.int32),
        mesh=mesh,
        scratch_types=[
            pltpu.VMEM((rows, _LANES), jnp.int32),
            pltpu.VMEM((crows, _LANES), jnp.int32),
            pltpu.VMEM((crows, _LANES), jnp.int32),
            pltpu.VMEM((_L,), jnp.int32),
            pltpu.SemaphoreType.DMA,
            pltpu.SemaphoreType.DMA,
        ],
        compiler_params=pltpu.CompilerParams(needs_layout_passes=False),
    )
    def hist_kernel(keys_hbm, bvec_hbm, out_hbm, hist_v, buf_a, buf_b,
                    bvec_v, sem_a, sem_b):
        wid = lax.axis_index("s") * _NC + lax.axis_index("c")
        base = wid * _WROWS
        pltpu.sync_copy(bvec_hbm, bvec_v)

        zeros = jnp.zeros((_L,), jnp.int32)

        @plsc.parallel_loop(0, rows)
        def _(i):
            for u in range(8):
                hist_v[i, pl.ds(u * _L, _L)] = zeros

        ones = jnp.ones((_L,), jnp.int32)
        bv = bvec_v[...]
        bufs = [buf_a, buf_b]
        sems = [sem_a, sem_b]
        cps = [None, None]
        cps[0] = pltpu.async_copy(
            keys_hbm.at[pl.ds(base, crows)], buf_a, sem_a)
        for c in range(nchunks):
            if c + 1 < nchunks:
                nb = (c + 1) % 2
                cps[nb] = pltpu.async_copy(
                    keys_hbm.at[pl.ds(base + (c + 1) * crows, crows)],
                    bufs[nb], sems[nb])
            cps[c % 2].wait()
            cur = bufs[c % 2]

            @plsc.parallel_loop(0, crows, unroll=2)
            def _(r, cur=cur):
                for u in range(_LANES // _L):
                    kv = cur[r, pl.ds(u * _L, _L)]
                    if level == 1:
                        sel = kv != 0
                        rr = jnp.clip(
                            lax.shift_right_logical(kv, 23) - _ROW0, 0, _R1 - 1)
                        col = lax.bitwise_and(
                            lax.shift_right_logical(kv, 16), _LANES - 1)
                        plsc.addupdate_scatter(
                            hist_v, [rr, col], ones, mask=sel)
                    else:
                        sel = lax.shift_right_logical(kv, 16) == bv
                        rr = lax.bitwise_and(
                            lax.shift_right_logical(kv, 7), _R2 - 1)
                        col = lax.bitwise_and(kv, _LANES - 1)
                        plsc.addupdate_scatter(
                            hist_v, [rr, col], ones, mask=sel)

        pltpu.sync_copy(hist_v, out_hbm.at[wid])

    return hist_kernel


@functools.lru_cache(maxsize=None)
def _make_select(rows, wblk, derive_k):
    nsteps = _NW // wblk

    def body(kk_ref, hist_ref, b_ref, cnt_ref, kk_out_ref, hacc_ref):
        w = pl.program_id(0)

        @pl.when(w == 0)
        def _():
            hacc_ref[...] = jnp.zeros((rows, _LANES), jnp.float32)

        hacc_ref[...] = hacc_ref[...] + jnp.sum(
            hist_ref[...].astype(jnp.float32), axis=0)

        @pl.when(w == nsteps - 1)
        def _():
            H = hacc_ref[...]
            if derive_k:
                kkf = jnp.floor(jnp.sum(H) * 3.0 / 10.0)
            else:
                kkf = kk_ref[0].astype(jnp.float32)
            kk_out_ref[...] = kkf.astype(jnp.int32).reshape(1, 1)
            rowsum = jnp.sum(H, axis=1, keepdims=True)
            ri = lax.broadcasted_iota(jnp.int32, (rows, rows), 0)
            qi = lax.broadcasted_iota(jnp.int32, (rows, rows), 1)
            tri = (qi > ri).astype(jnp.float32)
            rows_after = jnp.dot(tri, rowsum,
                                 preferred_element_type=jnp.float32)
            ci = lax.broadcasted_iota(jnp.int32, (_LANES, _LANES), 0)
            cj = lax.broadcasted_iota(jnp.int32, (_LANES, _LANES), 1)
            upp = (ci > cj).astype(jnp.float32)
            within = jnp.dot(H, upp, preferred_element_type=jnp.float32)
            above = rows_after + within
            found = (above < kkf) & (above + H >= kkf)
            binidx = (
                lax.broadcasted_iota(jnp.int32, (rows, _LANES), 0) * _LANES
                + lax.broadcasted_iota(jnp.int32, (rows, _LANES), 1)
            )
            b_ref[...] = jnp.sum(jnp.where(found, binidx, 0)).reshape(1, 1)
            cnt_ref[...] = (
                jnp.sum(jnp.where(found, above, 0.0))
                .astype(jnp.int32).reshape(1, 1)
            )

    def call(hist, kk):
        return pl.pallas_call(
            body,
            grid=(nsteps,),
            in_specs=[
                pl.BlockSpec(memory_space=pltpu.SMEM),
                pl.BlockSpec((wblk, rows, _LANES), lambda w: (w, 0, 0)),
            ],
            out_specs=[
                pl.BlockSpec((1, 1), lambda w: (0, 0)),
                pl.BlockSpec((1, 1), lambda w: (0, 0)),
                pl.BlockSpec((1, 1), lambda w: (0, 0)),
            ],
            out_shape=[
                jax.ShapeDtypeStruct((1, 1), jnp.int32),
                jax.ShapeDtypeStruct((1, 1), jnp.int32),
                jax.ShapeDtypeStruct((1, 1), jnp.int32),
            ],
            scratch_shapes=[pltpu.VMEM((rows, _LANES), jnp.float32)],
        )(kk, hist)

    return call


def _final_body(s_ref, keys_ref, out_ref, acc_ref):
    key = keys_ref[...]
    thr = s_ref[0]
    m = key > thr
    pt = lax.bitcast_convert_type(key, jnp.float32)
    pts = jnp.where(m, pt, jnp.float32(1.0))
    d = 1.0 - pts
    contrib = (d * d) * jnp.log(pts)

    @pl.when(pl.program_id(0) == 0)
    def _():
        acc_ref[...] = jnp.zeros((8, _LANES), jnp.float32)

    acc_ref[...] = acc_ref[...] + jnp.sum(
        contrib.reshape(_BLK_ROWS // 8, 8, _LANES), axis=0)

    @pl.when(pl.program_id(0) == _GRID - 1)
    def _():
        tv = jnp.full((1, 1), s_ref[0], jnp.int32)
        pt_t = lax.bitcast_convert_type(tv, jnp.float32)
        d_t = 1.0 - pt_t
        contrib_t = (d_t * d_t) * jnp.log(pt_t)
        ties = (s_ref[2] - s_ref[1]).astype(jnp.float32)
        kkf = s_ref[2].astype(jnp.float32)
        total = jnp.sum(acc_ref[...]).reshape(1, 1) + ties * contrib_t
        out_ref[...] = -total / kkf


def _final_call(scal, keys2d):
    return pl.pallas_call(
        _final_body,
        grid=(_GRID,),
        in_specs=[
            pl.BlockSpec(memory_space=pltpu.SMEM),
            pl.BlockSpec((_BLK_ROWS, _LANES), lambda i: (i, 0)),
        ],
        out_specs=pl.BlockSpec((1, 1), lambda i: (0, 0)),
        out_shape=jax.ShapeDtypeStruct((1, 1), jnp.float32),
        scratch_shapes=[pltpu.VMEM((8, _LANES), jnp.float32)],
    )(scal, keys2d)


def kernel(predicted, gts):
    pred3 = predicted.reshape(2, _ROWS, _LANES)
    gt2 = gts.reshape(_ROWS, _LANES) != 0

    keys2d = _keys_call(pred3, gt2)

    dummy = jnp.zeros((_L,), jnp.int32)
    zero1 = jnp.zeros((1,), jnp.int32)
    hist1 = _make_hist_kernel(1)(keys2d, dummy)
    b1, cnt_above, kkv = _make_select(_R1, 8, True)(hist1, zero1)
    kk = kkv[0, 0]
    top16 = b1[0, 0] + _TOP0

    bvec = jnp.full((_L,), top16, jnp.int32)
    hist2 = _make_hist_kernel(2)(keys2d, bvec)
    kk2 = kk - cnt_above[0, 0]
    t2, cnt2, _ = _make_select(_R2, 4, False)(hist2, kk2.reshape(1))

    thr = top16 * 65536 + t2[0, 0]
    cnt_gt = cnt_above[0, 0] + cnt2[0, 0]
    scal = jnp.stack([thr, cnt_gt, kk])
    out = _final_call(scal, keys2d)
    return out[0, 0]


# speedup vs baseline: 1.2121x; 1.2121x over previous
import functools

import jax
import jax.numpy as jnp
import numpy as np
from jax import lax
from jax.experimental import pallas as pl
from jax.experimental.pallas import tpu as pltpu
from jax.experimental.pallas import tpu_sc as plsc

_N = 2048 * 2048
_LANES = 128
_ROWS = _N // _LANES
_BLK_ROWS = 512
_GRID = _ROWS // _BLK_ROWS
_W_LO = float(np.float32(1e-5))
_W_HI = float(np.float32(1.0 - 1e-5))

_NC, _NS, _L = 2, 16, 16
_NW = _NC * _NS
_CROWS = 128
_WROWS = _ROWS // _NW
_NCHUNKS = _WROWS // _CROWS

_ROW0 = 110
_R1 = 32
_R2 = 512
_TOP0 = _ROW0 * _LANES


def _keys_body(pred_ref, gt_ref, keys_ref):
    p0 = pred_ref[0, :]
    p1 = pred_ref[1, :]
    m = gt_ref[0, 0, 0, :] != 0
    psel = jnp.where(m, p1, p0)
    pt = _W_LO * (p0 + p1) + (_W_HI - _W_LO) * psel + _W_LO
    key = lax.bitcast_convert_type(pt, jnp.int32)
    keys_ref[...] = jnp.where(m, key, 0).reshape(_BLK_ROWS, _LANES)


def _keys_call(predicted, gts):
    blk = (_BLK_ROWS, _LANES)
    return pl.pallas_call(
        _keys_body,
        grid=(_GRID,),
        in_specs=[
            pl.BlockSpec((2, _BLK_ROWS * _LANES), lambda i: (0, i)),
            pl.BlockSpec((1, 1, 1, _BLK_ROWS * _LANES),
                         lambda i: (0, 0, 0, i)),
        ],
        out_specs=pl.BlockSpec(blk, lambda i: (i, 0)),
        out_shape=jax.ShapeDtypeStruct((_ROWS, _LANES), jnp.int32),
    )(predicted, gts)


@functools.lru_cache(maxsize=None)
def _make_hist_kernel(level):
    mesh = plsc.VectorSubcoreMesh(
        core_axis_name="c", subcore_axis_name="s",
        num_cores=_NC, num_subcores=_NS,
    )
    rows = _R1 if level == 1 else _R2
    crows = 256 if level == 1 else _CROWS
    nchunks = _WROWS // crows

    @functools.partial(
        pl.kernel,
        out_type=jax.ShapeDtypeStruct((_NW, rows, _LANES), jnp.int32),
        mesh=mesh,
        scratch_types=[
            pltpu.VMEM((rows, _LANES), jnp.int32),
            pltpu.VMEM((crows, _LANES), jnp.int32),
            pltpu.VMEM((crows, _LANES), jnp.int32),
            pltpu.VMEM((_L,), jnp.int32),
            pltpu.SemaphoreType.DMA,
            pltpu.SemaphoreType.DMA,
        ],
        compiler_params=pltpu.CompilerParams(needs_layout_passes=False),
    )
    def hist_kernel(keys_hbm, bvec_hbm, out_hbm, hist_v, buf_a, buf_b,
                    bvec_v, sem_a, sem_b):
        wid = lax.axis_index("s") * _NC + lax.axis_index("c")
        base = wid * _WROWS
        pltpu.sync_copy(bvec_hbm, bvec_v)

        zeros = jnp.zeros((_L,), jnp.int32)

        @plsc.parallel_loop(0, rows)
        def _(i):
            for u in range(8):
                hist_v[i, pl.ds(u * _L, _L)] = zeros

        ones = jnp.ones((_L,), jnp.int32)
        bv = bvec_v[...]
        bufs = [buf_a, buf_b]
        sems = [sem_a, sem_b]
        cps = [None, None]
        cps[0] = pltpu.async_copy(
            keys_hbm.at[pl.ds(base, crows)], buf_a, sem_a)
        for c in range(nchunks):
            if c + 1 < nchunks:
                nb = (c + 1) % 2
                cps[nb] = pltpu.async_copy(
                    keys_hbm.at[pl.ds(base + (c + 1) * crows, crows)],
                    bufs[nb], sems[nb])
            cps[c % 2].wait()
            cur = bufs[c % 2]

            @plsc.parallel_loop(0, crows, unroll=2)
            def _(r, cur=cur):
                for u in range(_LANES // _L):
                    kv = cur[r, pl.ds(u * _L, _L)]
                    if level == 1:
                        sel = kv != 0
                        rr = jnp.clip(
                            lax.shift_right_logical(kv, 23) - _ROW0, 0, _R1 - 1)
                        col = lax.bitwise_and(
                            lax.shift_right_logical(kv, 16), _LANES - 1)
                        plsc.addupdate_scatter(
                            hist_v, [rr, col], ones, mask=sel)
                    else:
                        sel = lax.shift_right_logical(kv, 16) == bv
                        rr = lax.bitwise_and(
                            lax.shift_right_logical(kv, 7), _R2 - 1)
                        col = lax.bitwise_and(kv, _LANES - 1)
                        plsc.addupdate_scatter(
                            hist_v, [rr, col], ones, mask=sel)

        pltpu.sync_copy(hist_v, out_hbm.at[wid])

    return hist_kernel


@functools.lru_cache(maxsize=None)
def _make_select(rows, wblk, derive_k):
    nsteps = _NW // wblk

    def body(kk_ref, hist_ref, b_ref, cnt_ref, kk_out_ref, hacc_ref):
        w = pl.program_id(0)

        @pl.when(w == 0)
        def _():
            hacc_ref[...] = jnp.zeros((rows, _LANES), jnp.float32)

        hacc_ref[...] = hacc_ref[...] + jnp.sum(
            hist_ref[...].astype(jnp.float32), axis=0)

        @pl.when(w == nsteps - 1)
        def _():
            H = hacc_ref[...]
            if derive_k:
                kkf = jnp.floor(jnp.sum(H) * 3.0 / 10.0)
            else:
                kkf = kk_ref[0].astype(jnp.float32)
            kk_out_ref[...] = kkf.astype(jnp.int32).reshape(1, 1)
            rowsum = jnp.sum(H, axis=1, keepdims=True)
            ri = lax.broadcasted_iota(jnp.int32, (rows, rows), 0)
            qi = lax.broadcasted_iota(jnp.int32, (rows, rows), 1)
            tri = (qi > ri).astype(jnp.float32)
            rows_after = jnp.dot(tri, rowsum,
                                 preferred_element_type=jnp.float32)
            ci = lax.broadcasted_iota(jnp.int32, (_LANES, _LANES), 0)
            cj = lax.broadcasted_iota(jnp.int32, (_LANES, _LANES), 1)
            upp = (ci > cj).astype(jnp.float32)
            within = jnp.dot(H, upp, preferred_element_type=jnp.float32)
            above = rows_after + within
            found = (above < kkf) & (above + H >= kkf)
            binidx = (
                lax.broadcasted_iota(jnp.int32, (rows, _LANES), 0) * _LANES
                + lax.broadcasted_iota(jnp.int32, (rows, _LANES), 1)
            )
            b_ref[...] = jnp.sum(jnp.where(found, binidx, 0)).reshape(1, 1)
            cnt_ref[...] = (
                jnp.sum(jnp.where(found, above, 0.0))
                .astype(jnp.int32).reshape(1, 1)
            )

    def call(hist, kk):
        return pl.pallas_call(
            body,
            grid=(nsteps,),
            in_specs=[
                pl.BlockSpec(memory_space=pltpu.SMEM),
                pl.BlockSpec((wblk, rows, _LANES), lambda w: (w, 0, 0)),
            ],
            out_specs=[
                pl.BlockSpec((1, 1), lambda w: (0, 0)),
                pl.BlockSpec((1, 1), lambda w: (0, 0)),
                pl.BlockSpec((1, 1), lambda w: (0, 0)),
            ],
            out_shape=[
                jax.ShapeDtypeStruct((1, 1), jnp.int32),
                jax.ShapeDtypeStruct((1, 1), jnp.int32),
                jax.ShapeDtypeStruct((1, 1), jnp.int32),
            ],
            scratch_shapes=[pltpu.VMEM((rows, _LANES), jnp.float32)],
        )(kk, hist)

    return call


def _final_body(s_ref, keys_ref, out_ref, acc_ref):
    key = keys_ref[...]
    thr = s_ref[0]
    m = key > thr
    pt = lax.bitcast_convert_type(key, jnp.float32)
    pts = jnp.where(m, pt, jnp.float32(1.0))
    d = 1.0 - pts
    contrib = (d * d) * jnp.log(pts)

    @pl.when(pl.program_id(0) == 0)
    def _():
        acc_ref[...] = jnp.zeros((8, _LANES), jnp.float32)

    acc_ref[...] = acc_ref[...] + jnp.sum(
        contrib.reshape(_BLK_ROWS // 8, 8, _LANES), axis=0)

    @pl.when(pl.program_id(0) == _GRID - 1)
    def _():
        tv = jnp.full((1, 1), s_ref[0], jnp.int32)
        pt_t = lax.bitcast_convert_type(tv, jnp.float32)
        d_t = 1.0 - pt_t
        contrib_t = (d_t * d_t) * jnp.log(pt_t)
        ties = (s_ref[2] - s_ref[1]).astype(jnp.float32)
        kkf = s_ref[2].astype(jnp.float32)
        total = jnp.sum(acc_ref[...]).reshape(1, 1) + ties * contrib_t
        out_ref[...] = -total / kkf


def _final_call(scal, keys2d):
    return pl.pallas_call(
        _final_body,
        grid=(_GRID,),
        in_specs=[
            pl.BlockSpec(memory_space=pltpu.SMEM),
            pl.BlockSpec((_BLK_ROWS, _LANES), lambda i: (i, 0)),
        ],
        out_specs=pl.BlockSpec((1, 1), lambda i: (0, 0)),
        out_shape=jax.ShapeDtypeStruct((1, 1), jnp.float32),
        scratch_shapes=[pltpu.VMEM((8, _LANES), jnp.float32)],
    )(scal, keys2d)


def kernel(predicted, gts):
    keys2d = _keys_call(predicted, gts)

    dummy = jnp.zeros((_L,), jnp.int32)
    zero1 = jnp.zeros((1,), jnp.int32)
    hist1 = _make_hist_kernel(1)(keys2d, dummy)
    b1, cnt_above, kkv = _make_select(_R1, 8, True)(hist1, zero1)
    kk = kkv[0, 0]
    top16 = b1[0, 0] + _TOP0

    bvec = jnp.full((_L,), top16, jnp.int32)
    hist2 = _make_hist_kernel(2)(keys2d, bvec)
    kk2 = kk - cnt_above[0, 0]
    t2, cnt2, _ = _make_select(_R2, 4, False)(hist2, kk2.reshape(1))

    thr = top16 * 65536 + t2[0, 0]
    cnt_gt = cnt_above[0, 0] + cnt2[0, 0]
    scal = jnp.stack([thr, cnt_gt, kk])
    out = _final_call(scal, keys2d)
    return out[0, 0]


# speedup vs baseline: 1.3221x; 1.0907x over previous
import functools

import jax
import jax.numpy as jnp
import numpy as np
from jax import lax
from jax.experimental import pallas as pl
from jax.experimental.pallas import tpu as pltpu
from jax.experimental.pallas import tpu_sc as plsc

_N = 2048 * 2048
_LANES = 128
_ROWS = _N // _LANES
_BLK_ROWS = 512
_GRID = _ROWS // _BLK_ROWS
_W_LO = float(np.float32(1e-5))
_W_HI = float(np.float32(1.0 - 1e-5))

_NC, _NS, _L = 2, 16, 16
_NW = _NC * _NS
_CROWS = 128
_WROWS = _ROWS // _NW
_NCHUNKS = _WROWS // _CROWS

_ROW0 = 110
_R1 = 32
_R2 = 512
_TOP0 = _ROW0 * _LANES


def _keys_body(pred_ref, gt_ref, keys_ref):
    p0 = pred_ref[0, :]
    p1 = pred_ref[1, :]
    m = gt_ref[0, 0, 0, :] != 0
    psel = jnp.where(m, p1, p0)
    pt = _W_LO * (p0 + p1) + (_W_HI - _W_LO) * psel + _W_LO
    key = lax.bitcast_convert_type(pt, jnp.int32)
    keys_ref[...] = jnp.where(m, key, 0).reshape(_BLK_ROWS, _LANES)


def _keys_call(predicted, gts):
    blk = (_BLK_ROWS, _LANES)
    return pl.pallas_call(
        _keys_body,
        grid=(_GRID,),
        in_specs=[
            pl.BlockSpec((2, _BLK_ROWS * _LANES), lambda i: (0, i)),
            pl.BlockSpec((1, 1, 1, _BLK_ROWS * _LANES),
                         lambda i: (0, 0, 0, i)),
        ],
        out_specs=pl.BlockSpec(blk, lambda i: (i, 0)),
        out_shape=jax.ShapeDtypeStruct((_ROWS, _LANES), jnp.int32),
    )(predicted, gts)


@functools.lru_cache(maxsize=None)
def _make_hist_kernel(level):
    mesh = plsc.VectorSubcoreMesh(
        core_axis_name="c", subcore_axis_name="s",
        num_cores=_NC, num_subcores=_NS,
    )
    rows = _R1 if level == 1 else _R2
    crows = 256 if level == 1 else _CROWS
    nchunks = _WROWS // crows

    @functools.partial(
        pl.kernel,
        out_type=jax.ShapeDtypeStruct((_NW, rows, _LANES), jnp.int32),
        mesh=mesh,
        scratch_types=[
            pltpu.VMEM((rows, _LANES), jnp.int32),
            pltpu.VMEM((crows, _LANES), jnp.int32),
            pltpu.VMEM((crows, _LANES), jnp.int32),
            pltpu.VMEM((_L,), jnp.int32),
            pltpu.SemaphoreType.DMA,
            pltpu.SemaphoreType.DMA,
        ],
        compiler_params=pltpu.CompilerParams(needs_layout_passes=False),
    )
    def hist_kernel(keys_hbm, bvec_hbm, out_hbm, hist_v, buf_a, buf_b,
                    bvec_v, sem_a, sem_b):
        wid = lax.axis_index("s") * _NC + lax.axis_index("c")
        base = wid * _WROWS
        pltpu.sync_copy(bvec_hbm, bvec_v)

        zeros = jnp.zeros((_L,), jnp.int32)

        @plsc.parallel_loop(0, rows)
        def _(i):
            for u in range(8):
                hist_v[i, pl.ds(u * _L, _L)] = zeros

        ones = jnp.ones((_L,), jnp.int32)
        bv = bvec_v[...]
        bufs = [buf_a, buf_b]
        sems = [sem_a, sem_b]
        cps = [None, None]
        cps[0] = pltpu.async_copy(
            keys_hbm.at[pl.ds(base, crows)], buf_a, sem_a)
        for c in range(nchunks):
            if c + 1 < nchunks:
                nb = (c + 1) % 2
                cps[nb] = pltpu.async_copy(
                    keys_hbm.at[pl.ds(base + (c + 1) * crows, crows)],
                    bufs[nb], sems[nb])
            cps[c % 2].wait()
            cur = bufs[c % 2]

            @plsc.parallel_loop(0, crows, unroll=2)
            def _(r, cur=cur):
                for u in range(_LANES // _L):
                    kv = cur[r, pl.ds(u * _L, _L)]
                    if level == 1:
                        sel = kv != 0
                        rr = jnp.clip(
                            lax.shift_right_logical(kv, 23) - _ROW0, 0, _R1 - 1)
                        col = lax.bitwise_and(
                            lax.shift_right_logical(kv, 16), _LANES - 1)
                        plsc.addupdate_scatter(
                            hist_v, [rr, col], ones, mask=sel)
                    else:
                        sel = lax.shift_right_logical(kv, 16) == bv
                        rr = lax.bitwise_and(
                            lax.shift_right_logical(kv, 7), _R2 - 1)
                        col = lax.bitwise_and(kv, _LANES - 1)
                        plsc.addupdate_scatter(
                            hist_v, [rr, col], ones, mask=sel)

        pltpu.sync_copy(hist_v, out_hbm.at[wid])

    return hist_kernel


@functools.lru_cache(maxsize=None)
def _make_select(rows, wblk, derive_k):
    nsteps = _NW // wblk

    def body(kk_ref, hist_ref, b_ref, cnt_ref, kk_out_ref, hacc_ref):
        w = pl.program_id(0)

        @pl.when(w == 0)
        def _():
            hacc_ref[...] = jnp.zeros((rows, _LANES), jnp.float32)

        hacc_ref[...] = hacc_ref[...] + jnp.sum(
            hist_ref[...].astype(jnp.float32), axis=0)

        @pl.when(w == nsteps - 1)
        def _():
            H = hacc_ref[...]
            if derive_k:
                kkf = jnp.floor(jnp.sum(H) * 3.0 / 10.0)
            else:
                kkf = kk_ref[0].astype(jnp.float32)
            kk_out_ref[...] = kkf.astype(jnp.int32).reshape(1, 1)
            rowsum = jnp.sum(H, axis=1, keepdims=True)
            ri = lax.broadcasted_iota(jnp.int32, (rows, rows), 0)
            qi = lax.broadcasted_iota(jnp.int32, (rows, rows), 1)
            tri = (qi > ri).astype(jnp.float32)
            rows_after = jnp.dot(tri, rowsum,
                                 preferred_element_type=jnp.float32)
            ci = lax.broadcasted_iota(jnp.int32, (_LANES, _LANES), 0)
            cj = lax.broadcasted_iota(jnp.int32, (_LANES, _LANES), 1)
            upp = (ci > cj).astype(jnp.float32)
            within = jnp.dot(H, upp, preferred_element_type=jnp.float32)
            above = rows_after + within
            found = (above < kkf) & (above + H >= kkf)
            binidx = (
                lax.broadcasted_iota(jnp.int32, (rows, _LANES), 0) * _LANES
                + lax.broadcasted_iota(jnp.int32, (rows, _LANES), 1)
            )
            b_ref[...] = jnp.sum(jnp.where(found, binidx, 0)).reshape(1, 1)
            cnt_ref[...] = (
                jnp.sum(jnp.where(found, above, 0.0))
                .astype(jnp.int32).reshape(1, 1)
            )

    def call(hist, kk):
        return pl.pallas_call(
            body,
            grid=(nsteps,),
            in_specs=[
                pl.BlockSpec(memory_space=pltpu.SMEM),
                pl.BlockSpec((wblk, rows, _LANES), lambda w: (w, 0, 0)),
            ],
            out_specs=[
                pl.BlockSpec((1, 1), lambda w: (0, 0)),
                pl.BlockSpec((1, 1), lambda w: (0, 0)),
                pl.BlockSpec((1, 1), lambda w: (0, 0)),
            ],
            out_shape=[
                jax.ShapeDtypeStruct((1, 1), jnp.int32),
                jax.ShapeDtypeStruct((1, 1), jnp.int32),
                jax.ShapeDtypeStruct((1, 1), jnp.int32),
            ],
            scratch_shapes=[pltpu.VMEM((rows, _LANES), jnp.float32)],
        )(kk, hist)

    return call


_D_ROWS = 1024
_D_GRID = _ROWS // _D_ROWS


def _final_body(s_ref, keys_ref, out_ref, acc_ref):
    key = keys_ref[...]
    thr = s_ref[0]
    m = key > thr
    pt = lax.bitcast_convert_type(key, jnp.float32)
    pts = jnp.where(m, pt, jnp.float32(1.0))
    d = 1.0 - pts
    contrib = (d * d) * jnp.log(pts)

    @pl.when(pl.program_id(0) == 0)
    def _():
        acc_ref[...] = jnp.zeros((8, _LANES), jnp.float32)

    acc_ref[...] = acc_ref[...] + jnp.sum(
        contrib.reshape(_D_ROWS // 8, 8, _LANES), axis=0)

    @pl.when(pl.program_id(0) == _D_GRID - 1)
    def _():
        tv = jnp.full((1, 1), s_ref[0], jnp.int32)
        pt_t = lax.bitcast_convert_type(tv, jnp.float32)
        d_t = 1.0 - pt_t
        contrib_t = (d_t * d_t) * jnp.log(pt_t)
        ties = (s_ref[2] - s_ref[1]).astype(jnp.float32)
        kkf = s_ref[2].astype(jnp.float32)
        total = jnp.sum(acc_ref[...]).reshape(1, 1) + ties * contrib_t
        out_ref[...] = -total / kkf


def _final_call(scal, keys2d):
    return pl.pallas_call(
        _final_body,
        grid=(_D_GRID,),
        in_specs=[
            pl.BlockSpec(memory_space=pltpu.SMEM),
            pl.BlockSpec((_D_ROWS, _LANES), lambda i: (i, 0)),
        ],
        out_specs=pl.BlockSpec((1, 1), lambda i: (0, 0)),
        out_shape=jax.ShapeDtypeStruct((1, 1), jnp.float32),
        scratch_shapes=[pltpu.VMEM((8, _LANES), jnp.float32)],
    )(scal, keys2d)


def kernel(predicted, gts):
    keys2d = _keys_call(predicted, gts)

    dummy = jnp.zeros((_L,), jnp.int32)
    zero1 = jnp.zeros((1,), jnp.int32)
    hist1 = _make_hist_kernel(1)(keys2d, dummy)
    b1, cnt_above, kkv = _make_select(_R1, 8, True)(hist1, zero1)
    kk = kkv[0, 0]
    top16 = b1[0, 0] + _TOP0

    bvec = jnp.full((_L,), top16, jnp.int32)
    hist2 = _make_hist_kernel(2)(keys2d, bvec)
    kk2 = kk - cnt_above[0, 0]
    t2, cnt2, _ = _make_select(_R2, 4, False)(hist2, kk2.reshape(1))

    thr = top16 * 65536 + t2[0, 0]
    cnt_gt = cnt_above[0, 0] + cnt2[0, 0]
    scal = jnp.stack([thr, cnt_gt, kk])
    out = _final_call(scal, keys2d)
    return out[0, 0]
